# baseline (speedup 1.0000x reference)
import jax
import jax.numpy as jnp
from jax import lax
from jax.experimental import pallas as pl
from jax.experimental.pallas import tpu as pltpu

_TILE_N = 1024
_K_SLOTS = 4


def _make_kernel(k_slots):
    def body(*refs):
        w_ref, b_ref = refs[0], refs[1]
        h_refs = refs[2:2 + k_slots]
        o_ref = refs[2 + k_slots]
        w = w_ref[...]
        b = b_ref[0, 0]
        accs = [
            lax.dot_general(
                w, h_ref[...],
                dimension_numbers=(((1,), (1,)), ((), ())),
                preferred_element_type=jnp.float32,
            )
            for h_ref in h_refs
        ]
        o_ref[...] = (jnp.concatenate(accs, axis=0) + b).astype(o_ref.dtype)
    return body


def kernel(h, weight, bias):
    n, d = h.shape
    b2 = bias.reshape(1, 1).astype(jnp.float32)
    w = weight.astype(jnp.float32)

    tile_n = _TILE_N
    k = _K_SLOTS
    if n % (k * tile_n) != 0:
        tile_n1 = n if n <= 1024 else 1024
        num_tiles = pl.cdiv(n, tile_n1)
        out_row = pl.pallas_call(
            lambda w_ref, b_ref, h_ref, o_ref: o_ref.__setitem__(
                ...,
                (lax.dot_general(w_ref[...], h_ref[...],
                                 dimension_numbers=(((1,), (1,)), ((), ())),
                                 preferred_element_type=jnp.float32)
                 + b_ref[0, 0]).astype(o_ref.dtype)),
            out_shape=jax.ShapeDtypeStruct((1, num_tiles * tile_n1), h.dtype),
            grid=(num_tiles,),
            in_specs=[
                pl.BlockSpec((1, d), lambda i: (0, 0)),
                pl.BlockSpec(memory_space=pltpu.SMEM),
                pl.BlockSpec((tile_n1, d), lambda i: (i, 0)),
            ],
            out_specs=pl.BlockSpec((1, tile_n1), lambda i: (0, i)),
            compiler_params=pltpu.CompilerParams(
                dimension_semantics=("parallel",)),
        )(w, b2, h)
        return out_row[0, :n].reshape(n, 1)

    num_steps = n // (k * tile_n)

    def h_spec(slot):
        return pl.BlockSpec((tile_n, d),
                            lambda i, s=slot: (s * num_steps + i, 0))

    bytes_accessed = n * d * h.dtype.itemsize + d * 4 + n * h.dtype.itemsize
    cost = pl.CostEstimate(flops=2 * n * d, transcendentals=0,
                           bytes_accessed=bytes_accessed)

    outk = pl.pallas_call(
        _make_kernel(k),
        out_shape=jax.ShapeDtypeStruct((k, n // k), h.dtype),
        grid=(num_steps,),
        in_specs=[
            pl.BlockSpec((1, d), lambda i: (0, 0)),
            pl.BlockSpec(memory_space=pltpu.SMEM),
        ] + [h_spec(s) for s in range(k)],
        out_specs=pl.BlockSpec((k, tile_n), lambda i: (0, i)),
        compiler_params=pltpu.CompilerParams(
            dimension_semantics=("parallel",),
        ),
        cost_estimate=cost,
    )(w, b2, *([h] * k))

    return outk.reshape(n, 1)


# speedup vs baseline: 1.0020x; 1.0020x over previous
import jax
import jax.numpy as jnp
from jax import lax
from jax.experimental import pallas as pl
from jax.experimental.pallas import tpu as pltpu

_TILE_N = 2048
_K_SLOTS = 2


def _make_kernel(k_slots):
    def body(*refs):
        w_ref, b_ref = refs[0], refs[1]
        h_refs = refs[2:2 + k_slots]
        o_ref = refs[2 + k_slots]
        w = w_ref[...]
        b = b_ref[0, 0]
        accs = [
            lax.dot_general(
                w, h_ref[...],
                dimension_numbers=(((1,), (1,)), ((), ())),
                preferred_element_type=jnp.float32,
            )
            for h_ref in h_refs
        ]
        o_ref[...] = (jnp.concatenate(accs, axis=0) + b).astype(o_ref.dtype)
    return body


def kernel(h, weight, bias):
    n, d = h.shape
    b2 = bias.reshape(1, 1).astype(jnp.float32)
    w = weight.astype(jnp.float32)

    tile_n = _TILE_N
    k = _K_SLOTS
    if n % (k * tile_n) != 0:
        tile_n1 = n if n <= 1024 else 1024
        num_tiles = pl.cdiv(n, tile_n1)
        out_row = pl.pallas_call(
            lambda w_ref, b_ref, h_ref, o_ref: o_ref.__setitem__(
                ...,
                (lax.dot_general(w_ref[...], h_ref[...],
                                 dimension_numbers=(((1,), (1,)), ((), ())),
                                 preferred_element_type=jnp.float32)
                 + b_ref[0, 0]).astype(o_ref.dtype)),
            out_shape=jax.ShapeDtypeStruct((1, num_tiles * tile_n1), h.dtype),
            grid=(num_tiles,),
            in_specs=[
                pl.BlockSpec((1, d), lambda i: (0, 0)),
                pl.BlockSpec(memory_space=pltpu.SMEM),
                pl.BlockSpec((tile_n1, d), lambda i: (i, 0)),
            ],
            out_specs=pl.BlockSpec((1, tile_n1), lambda i: (0, i)),
            compiler_params=pltpu.CompilerParams(
                dimension_semantics=("parallel",)),
        )(w, b2, h)
        return out_row[0, :n].reshape(n, 1)

    num_steps = n // (k * tile_n)

    def h_spec(slot):
        return pl.BlockSpec((tile_n, d),
                            lambda i, s=slot: (s * num_steps + i, 0))

    bytes_accessed = n * d * h.dtype.itemsize + d * 4 + n * h.dtype.itemsize
    cost = pl.CostEstimate(flops=2 * n * d, transcendentals=0,
                           bytes_accessed=bytes_accessed)

    outk = pl.pallas_call(
        _make_kernel(k),
        out_shape=jax.ShapeDtypeStruct((k, n // k), h.dtype),
        grid=(num_steps,),
        in_specs=[
            pl.BlockSpec((1, d), lambda i: (0, 0)),
            pl.BlockSpec(memory_space=pltpu.SMEM),
        ] + [h_spec(s) for s in range(k)],
        out_specs=pl.BlockSpec((k, tile_n), lambda i: (0, i)),
        compiler_params=pltpu.CompilerParams(
            dimension_semantics=("parallel",),
        ),
        cost_estimate=cost,
    )(w, b2, *([h] * k))

    return outk.reshape(n, 1)
